# baseline (speedup 1.0000x reference)
import functools

import jax
import jax.numpy as jnp
from jax import lax
from jax.experimental import pallas as pl
from jax.experimental.pallas import tpu as pltpu
from jax.experimental.pallas import tpu_sc as plsc

N, E, D, DE, H, G = 10000, 320000, 128, 16, 16, 64

NC, NS, L = 2, 16, 16
HD = D // NC
EW = E // NS
C = 32
NCHUNK = EW // C
NP = 10240
RPS = NP // NS
ZR = 80
KV = HD // L
_STAGE = 4


def _sigmoid(z):
    return 1.0 / (1.0 + jnp.exp(-z))


def _softplus(z):
    t = jnp.exp(-jnp.abs(z))
    r = t / (2.0 + t)
    r2 = r * r
    p = 2.0 + r2 * (2.0 / 3.0 + r2 * (2.0 / 5.0 + r2 * (2.0 / 7.0 + r2 * (2.0 / 9.0))))
    return jnp.maximum(z, 0.0) + r * p


def _edge_kernel(td_hbm, us_hbm, ep_hbm, dsti_hbm, srci_hbm, out_hbm,
                 dst_v, dsta_v, srca_v, td_r, us_r, ep_r, m_r, zbuf, acc,
                 sem0, sem1, sem2):
    c = lax.axis_index("c")
    s = lax.axis_index("s")
    coff = c * N

    def zb(i, carry):
        for k in range(KV):
            zbuf[i, pl.ds(k * L, L)] = jnp.zeros((L,), jnp.float32)
        return carry
    lax.fori_loop(0, ZR, zb, 0)
    for j in range(RPS // ZR):
        pltpu.sync_copy(zbuf, acc.at[pl.ds(s * RPS + j * ZR, ZR)])
    plsc.subcore_barrier()

    @pl.loop(0, NCHUNK)
    def chunk(i):
        base = s * EW + i * C
        pltpu.sync_copy(dsti_hbm.at[pl.ds(base, C)], dst_v)
        pltpu.sync_copy(srci_hbm.at[pl.ds(base, C)], srca_v)
        for k in range(C // L):
            sl = pl.ds(k * L, L)
            dsta_v[sl] = dst_v[sl] + coff
            srca_v[sl] = srca_v[sl] + coff
        cp0 = pltpu.async_copy(td_hbm.at[dsta_v], td_r, sem0)
        cp1 = pltpu.async_copy(us_hbm.at[srca_v], us_r, sem1)
        cp2 = pltpu.async_copy(ep_hbm.at[pl.ds(c * E + base, C)], ep_r, sem2)
        cp0.wait()
        cp1.wait()
        cp2.wait()

        for r in range(C):
            for k in range(KV):
                lo = pl.ds(k * L, L)
                hi = pl.ds(HD + k * L, L)
                zf = td_r[r, lo] + us_r[r, lo] + ep_r[r, lo]
                zs = td_r[r, hi] + us_r[r, hi] + ep_r[r, hi]
                m_r[r, lo] = _sigmoid(zf) * _softplus(zs)

        pltpu.sync_copy(m_r, acc.at[dst_v], add=True)

    plsc.subcore_barrier()
    pltpu.sync_copy(acc.at[pl.ds(s * RPS, RPS)],
                    out_hbm.at[pl.ds(c * NP + s * RPS, RPS)])


@functools.partial(
    pl.kernel,
    mesh=plsc.VectorSubcoreMesh(core_axis_name="c", subcore_axis_name="s"),
    out_type=jax.ShapeDtypeStruct((NC * NP, HD), jnp.float32),
    scratch_types=[
        pltpu.VMEM((C,), jnp.int32),
        pltpu.VMEM((C,), jnp.int32),
        pltpu.VMEM((C,), jnp.int32),
        pltpu.VMEM((C, D), jnp.float32),
        pltpu.VMEM((C, D), jnp.float32),
        pltpu.VMEM((C, D), jnp.float32),
        pltpu.VMEM((C, HD), jnp.float32),
        pltpu.VMEM((ZR, HD), jnp.float32),
        pltpu.VMEM_SHARED((NP, HD), jnp.float32),
        pltpu.SemaphoreType.DMA,
        pltpu.SemaphoreType.DMA,
        pltpu.SemaphoreType.DMA,
    ],
)
def _edge_pass(td, us, ep, dsti, srci, out, *rest):
    _edge_kernel(td, us, ep, dsti, srci, out, *rest)


def _split_cols(Wf, Ws, lo, hi):
    return [jnp.concatenate([Wf[lo:hi, c * HD:(c + 1) * HD],
                             Ws[lo:hi, c * HD:(c + 1) * HD]], axis=1)
            for c in range(NC)]


def _layer(h, srci, dsti, ep, Wf, bf, Ws, bs):
    wtd = _split_cols(Wf, Ws, 0, D)
    wus = _split_cols(Wf, Ws, D, 2 * D)
    bc = [jnp.concatenate([bf[c * HD:(c + 1) * HD], bs[c * HD:(c + 1) * HD]])
          for c in range(NC)]
    td = jnp.concatenate([h @ wtd[c] + bc[c] for c in range(NC)])
    us = jnp.concatenate([h @ wus[c] for c in range(NC)])
    part = _edge_pass(td, us, ep, dsti, srci)
    return h + jnp.concatenate([part[:N], part[NP:NP + N]], axis=1)


def kernel(x, edge_index, edge_attr, batch, Wf1, bf1, Ws1, bs1,
           Wf2, bf2, Ws2, bs2, W1, b1, W2, b2):
    srci, dsti = edge_index[0], edge_index[1]
    wep1 = _split_cols(Wf1, Ws1, 2 * D, 2 * D + DE)
    wep2 = _split_cols(Wf2, Ws2, 2 * D, 2 * D + DE)
    ep1 = jnp.concatenate([edge_attr @ wep1[c] for c in range(NC)])
    ep2 = jnp.concatenate([edge_attr @ wep2[c] for c in range(NC)])
    h = _layer(x, srci, dsti, ep1, Wf1, bf1, Ws1, bs1)
    h = _layer(h, srci, dsti, ep2, Wf2, bf2, Ws2, bs2)
    onehot = (batch[:, None] == jnp.arange(G)[None, :]).astype(jnp.float32)
    sums = onehot.T @ h
    counts = jnp.sum(onehot, axis=0)[:, None]
    pooled = sums / jnp.maximum(counts, 1.0)
    return jax.nn.relu(pooled @ W1 + b1) @ W2 + b2
